# Initial kernel scaffold; baseline (speedup 1.0000x reference)
import jax
import jax.numpy as jnp
from jax.experimental import pallas as pl


def kernel(x, edge_index, pos_embedding, Wp, Wc, bc):
    raise NotImplementedError("write your pallas kernel here")


# speedup vs baseline: 19.5472x; 19.5472x over previous
import functools

import jax
import jax.numpy as jnp
from jax import lax
from jax.experimental import pallas as pl
from jax.experimental.pallas import tpu as pltpu
from jax.experimental.pallas import tpu_sc as plsc

N = 10000
E = 320000
CH = 256
HALF = 128
POS = 8

NC = 2
NS = 16
NW = NC * NS
CHUNK = 128
NCHUNK = 80
EPW = CHUNK * NCHUNK
EPAD = EPW * NW
R = 10240
RPT = R // NS

BN = 1024
NPAD = R
GRID = NPAD // BN


def _deg_body(dstp_hbm, zeros1_hbm, ones_hbm, degp_hbm, dst_v, ones_v, deg_sp, sem):
    c = lax.axis_index("c")
    s = lax.axis_index("s")
    wid = s * NC + c
    pltpu.sync_copy(zeros1_hbm, deg_sp.at[pl.ds(s * RPT, RPT)])
    pltpu.sync_copy(ones_hbm, ones_v)
    pltpu.sync_copy(dstp_hbm.at[wid], dst_v)
    plsc.subcore_barrier()

    def body(j, carry):
        pltpu.sync_copy(ones_v, deg_sp.at[dst_v.at[j]], add=True)
        return carry

    lax.fori_loop(0, NCHUNK, body, 0)
    plsc.subcore_barrier()
    pltpu.sync_copy(deg_sp.at[pl.ds(s * RPT, RPT)],
                    degp_hbm.at[c, pl.ds(s * RPT, RPT)])


def _sc_degree(dstp, zeros1, ones):
    mesh = plsc.VectorSubcoreMesh(core_axis_name="c", subcore_axis_name="s")
    return pl.kernel(
        _deg_body,
        out_type=jax.ShapeDtypeStruct((NC, R), jnp.float32),
        mesh=mesh,
        scratch_types=[
            pltpu.VMEM((NCHUNK, CHUNK), jnp.int32),
            pltpu.VMEM((CHUNK,), jnp.float32),
            pltpu.VMEM_SHARED((R,), jnp.float32),
            pltpu.SemaphoreType.DMA,
        ],
    )(dstp, zeros1, ones)


def _msg_body(g0_hbm, g1_hbm, srcp_hbm, dstp_hbm, zrows_hbm, acc0_hbm, acc1_hbm,
              src_v, dst_v, rows_v, acc_sp, sem):
    c = lax.axis_index("c")
    s = lax.axis_index("s")
    wid = s * NC + c
    pltpu.sync_copy(srcp_hbm.at[wid], src_v)
    pltpu.sync_copy(dstp_hbm.at[wid], dst_v)
    for g_hbm, acc_hbm in ((g0_hbm, acc0_hbm), (g1_hbm, acc1_hbm)):
        pltpu.sync_copy(zrows_hbm, acc_sp.at[pl.ds(s * RPT, RPT)])
        plsc.subcore_barrier()

        def body(j, carry):
            pltpu.async_copy(g_hbm.at[src_v.at[j]], rows_v, sem).wait()
            pltpu.sync_copy(rows_v, acc_sp.at[dst_v.at[j]], add=True)
            return carry

        lax.fori_loop(0, NCHUNK, body, 0)
        plsc.subcore_barrier()
        pltpu.sync_copy(acc_sp.at[pl.ds(s * RPT, RPT)],
                        acc_hbm.at[c, pl.ds(s * RPT, RPT)])
        plsc.subcore_barrier()


def _sc_messages(g0, g1, srcp, dstp, zrows):
    mesh = plsc.VectorSubcoreMesh(core_axis_name="c", subcore_axis_name="s")
    return pl.kernel(
        _msg_body,
        out_type=(
            jax.ShapeDtypeStruct((NC, R, HALF), jnp.float32),
            jax.ShapeDtypeStruct((NC, R, HALF), jnp.float32),
        ),
        mesh=mesh,
        scratch_types=[
            pltpu.VMEM((NCHUNK, CHUNK), jnp.int32),
            pltpu.VMEM((NCHUNK, CHUNK), jnp.int32),
            pltpu.VMEM((CHUNK, HALF), jnp.float32),
            pltpu.VMEM_SHARED((R, HALF), jnp.float32),
            pltpu.SemaphoreType.DMA,
        ],
    )(g0, g1, srcp, dstp, zrows)


def _dense_body(x_ref, pos_ref, wpT_ref, wcT_ref, degs_ref, g0_ref, g1_ref):
    y = jnp.dot(x_ref[...], wpT_ref[...], preferred_element_type=jnp.float32)
    pos = pos_ref[...]
    acc = jnp.zeros((BN, CH), dtype=jnp.float32)
    for p in range(POS):
        acc = acc + y[:, p * CH:(p + 1) * CH] * pos[:, p:p + 1]
    h3 = jnp.dot(acc, wcT_ref[...], preferred_element_type=jnp.float32)
    deg = degs_ref[:, 0] + degs_ref[:, 1] + 1.0
    dinv = lax.rsqrt(deg)
    g = h3 * dinv[:, None]
    g0_ref[...] = g[:, :HALF]
    g1_ref[...] = g[:, HALF:]


def _tc_dense(xp, posp, wpT, wcT, degsT):
    return pl.pallas_call(
        _dense_body,
        grid=(GRID,),
        in_specs=[
            pl.BlockSpec((BN, CH), lambda i: (i, 0)),
            pl.BlockSpec((BN, POS), lambda i: (i, 0)),
            pl.BlockSpec((CH, CH * POS), lambda i: (0, 0)),
            pl.BlockSpec((CH, CH), lambda i: (0, 0)),
            pl.BlockSpec((BN, NC), lambda i: (i, 0)),
        ],
        out_specs=(
            pl.BlockSpec((BN, HALF), lambda i: (i, 0)),
            pl.BlockSpec((BN, HALF), lambda i: (i, 0)),
        ),
        out_shape=(
            jax.ShapeDtypeStruct((NPAD, HALF), jnp.float32),
            jax.ShapeDtypeStruct((NPAD, HALF), jnp.float32),
        ),
    )(xp, posp, wpT, wcT, degsT)


def _final_body(acc0_ref, acc1_ref, g0_ref, g1_ref, degs_ref, bc_ref, out_ref):
    deg = degs_ref[:, 0] + degs_ref[:, 1] + 1.0
    dinv = lax.rsqrt(deg)[:, None]
    s0 = (acc0_ref[0] + acc0_ref[1] + g0_ref[...]) * dinv
    s1 = (acc1_ref[0] + acc1_ref[1] + g1_ref[...]) * dinv
    bc = bc_ref[...]
    out_ref[:, :HALF] = s0 + bc[:, :HALF]
    out_ref[:, HALF:] = s1 + bc[:, HALF:]


def _tc_final(acc0, acc1, g0, g1, degsT, bc2):
    return pl.pallas_call(
        _final_body,
        grid=(GRID,),
        in_specs=[
            pl.BlockSpec((NC, BN, HALF), lambda i: (0, i, 0)),
            pl.BlockSpec((NC, BN, HALF), lambda i: (0, i, 0)),
            pl.BlockSpec((BN, HALF), lambda i: (i, 0)),
            pl.BlockSpec((BN, HALF), lambda i: (i, 0)),
            pl.BlockSpec((BN, NC), lambda i: (i, 0)),
            pl.BlockSpec((1, CH), lambda i: (0, 0)),
        ],
        out_specs=pl.BlockSpec((BN, CH), lambda i: (i, 0)),
        out_shape=jax.ShapeDtypeStruct((NPAD, CH), jnp.float32),
    )(acc0, acc1, g0, g1, degsT, bc2)


def kernel(x, edge_index, pos_embedding, Wp, Wc, bc):
    src = edge_index[0]
    dst = edge_index[1]
    npad = EPAD - E
    pad_ids = jnp.arange(npad, dtype=jnp.int32)
    srcp = jnp.concatenate([src, pad_ids % N]).reshape(NW, NCHUNK, CHUNK)
    dstp = jnp.concatenate([dst, N + pad_ids % (R - N)]).reshape(NW, NCHUNK, CHUNK)

    zeros1 = jnp.zeros((RPT,), jnp.float32)
    ones = jnp.ones((CHUNK,), jnp.float32)
    zrows = jnp.zeros((RPT, HALF), jnp.float32)

    degp = _sc_degree(dstp, zeros1, ones)
    degsT = degp.T

    xp = jnp.zeros((NPAD, CH), x.dtype).at[:N].set(x)
    posp = jnp.zeros((NPAD, POS), pos_embedding.dtype).at[:N].set(pos_embedding)
    g0, g1 = _tc_dense(xp, posp, Wp.T, Wc.T, degsT)

    acc0, acc1 = _sc_messages(g0, g1, srcp, dstp, zrows)

    out = _tc_final(acc0, acc1, g0, g1, degsT, bc.reshape(1, CH))
    return out[:N]


# speedup vs baseline: 25.5305x; 1.3061x over previous
import jax
import jax.numpy as jnp
from jax import lax
from jax.experimental import pallas as pl
from jax.experimental.pallas import tpu as pltpu
from jax.experimental.pallas import tpu_sc as plsc

N = 10000
E = 320000
CH = 256
HALF = 128
POS = 8

NC = 2
NS = 16
CHUNK = 128
NCH = 160
EPAD = NS * NCH * CHUNK
R = 10240
RPT = R // NS
DCH = NCH // NC

BN = 1024
NPAD = R
GRID = NPAD // BN


def _deg_body(dstp_hbm, zeros1_hbm, ones_hbm, degp_hbm, dst_v, ones_v, deg_sp, sem):
    c = lax.axis_index("c")
    s = lax.axis_index("s")
    pltpu.sync_copy(zeros1_hbm, deg_sp.at[pl.ds(s * RPT, RPT)])
    pltpu.sync_copy(ones_hbm, ones_v)
    pltpu.sync_copy(dstp_hbm.at[s, pl.ds(c * DCH, DCH)], dst_v)
    plsc.subcore_barrier()

    def body(j, carry):
        pltpu.sync_copy(ones_v, deg_sp.at[dst_v.at[j]], add=True)
        return carry

    lax.fori_loop(0, DCH, body, 0)
    plsc.subcore_barrier()
    pltpu.sync_copy(deg_sp.at[pl.ds(s * RPT, RPT)],
                    degp_hbm.at[c, pl.ds(s * RPT, RPT)])


def _sc_degree(dstp, zeros1, ones):
    mesh = plsc.VectorSubcoreMesh(core_axis_name="c", subcore_axis_name="s")
    return pl.kernel(
        _deg_body,
        out_type=jax.ShapeDtypeStruct((NC, R), jnp.float32),
        mesh=mesh,
        scratch_types=[
            pltpu.VMEM((DCH, CHUNK), jnp.int32),
            pltpu.VMEM((CHUNK,), jnp.float32),
            pltpu.VMEM_SHARED((R,), jnp.float32),
            pltpu.SemaphoreType.DMA,
        ],
    )(dstp, zeros1, ones)


def _msg_body(g2_hbm, srcp_hbm, dstp_hbm, zrows_hbm, acc_hbm,
              src_a, src_b, dst_a, dst_b, rows_a, rows_b, acc_sp,
              sem_ia, sem_ib, sem_ra, sem_rb):
    c = lax.axis_index("c")
    s = lax.axis_index("s")
    pltpu.sync_copy(zrows_hbm, acc_sp.at[pl.ds(s * RPT, RPT)])
    plsc.subcore_barrier()

    srcs = (src_a, src_b)
    dsts = (dst_a, dst_b)
    rows = (rows_a, rows_b)
    isems = (sem_ia, sem_ib)
    rsems = (sem_ra, sem_rb)

    def fetch_idx(chunk, slot):
        pltpu.async_copy(srcp_hbm.at[c, s, chunk], srcs[slot], isems[slot])
        pltpu.async_copy(dstp_hbm.at[s, chunk], dsts[slot], isems[slot])

    def wait_idx(slot):
        pltpu.make_async_copy(srcp_hbm.at[c, s, 0], srcs[slot], isems[slot]).wait()
        pltpu.make_async_copy(dstp_hbm.at[s, 0], dsts[slot], isems[slot]).wait()

    def wait_rows(slot):
        pltpu.make_async_copy(g2_hbm.at[srcs[slot]], rows[slot], rsems[slot]).wait()

    fetch_idx(0, 0)
    wait_idx(0)
    pltpu.async_copy(g2_hbm.at[src_a], rows_a, sem_ra)
    fetch_idx(1, 1)

    def outer(j0, carry):
        for b in range(2):
            j = j0 * 2 + b
            nb = 1 - b
            wait_idx(nb)
            wait_rows(b)
            pltpu.async_copy(g2_hbm.at[srcs[nb]], rows[nb], rsems[nb])
            pltpu.sync_copy(rows[b], acc_sp.at[dsts[b]], add=True)
            fetch_idx(jnp.minimum(j + 2, NCH - 1), b)
        return carry

    lax.fori_loop(0, NCH // 2, outer, 0)
    wait_rows(0)
    wait_idx(1)

    plsc.subcore_barrier()
    pltpu.sync_copy(acc_sp.at[pl.ds(s * RPT, RPT)],
                    acc_hbm.at[c, pl.ds(s * RPT, RPT)])


def _sc_messages(g2, srcp2, dstp, zrows):
    mesh = plsc.VectorSubcoreMesh(core_axis_name="c", subcore_axis_name="s")
    return pl.kernel(
        _msg_body,
        out_type=jax.ShapeDtypeStruct((NC, R, HALF), jnp.float32),
        mesh=mesh,
        scratch_types=[
            pltpu.VMEM((CHUNK,), jnp.int32),
            pltpu.VMEM((CHUNK,), jnp.int32),
            pltpu.VMEM((CHUNK,), jnp.int32),
            pltpu.VMEM((CHUNK,), jnp.int32),
            pltpu.VMEM((CHUNK, HALF), jnp.float32),
            pltpu.VMEM((CHUNK, HALF), jnp.float32),
            pltpu.VMEM_SHARED((R, HALF), jnp.float32),
            pltpu.SemaphoreType.DMA,
            pltpu.SemaphoreType.DMA,
            pltpu.SemaphoreType.DMA,
            pltpu.SemaphoreType.DMA,
        ],
    )(g2, srcp2, dstp, zrows)


def _dense_body(x_ref, pos_ref, wpT_ref, wcT_ref, degs_ref, g_ref):
    y = jnp.dot(x_ref[...], wpT_ref[...], preferred_element_type=jnp.float32)
    pos = pos_ref[...]
    acc = jnp.zeros((BN, CH), dtype=jnp.float32)
    for p in range(POS):
        acc = acc + y[:, p * CH:(p + 1) * CH] * pos[:, p:p + 1]
    h3 = jnp.dot(acc, wcT_ref[...], preferred_element_type=jnp.float32)
    deg = degs_ref[:, 0] + degs_ref[:, 1] + 1.0
    dinv = lax.rsqrt(deg)
    g = h3 * dinv[:, None]
    g_ref[0] = g[:, :HALF]
    g_ref[1] = g[:, HALF:]


def _tc_dense(xp, posp, wpT, wcT, degsT):
    return pl.pallas_call(
        _dense_body,
        grid=(GRID,),
        in_specs=[
            pl.BlockSpec((BN, CH), lambda i: (i, 0)),
            pl.BlockSpec((BN, POS), lambda i: (i, 0)),
            pl.BlockSpec((CH, CH * POS), lambda i: (0, 0)),
            pl.BlockSpec((CH, CH), lambda i: (0, 0)),
            pl.BlockSpec((BN, NC), lambda i: (i, 0)),
        ],
        out_specs=pl.BlockSpec((NC, BN, HALF), lambda i: (0, i, 0)),
        out_shape=jax.ShapeDtypeStruct((NC, NPAD, HALF), jnp.float32),
    )(xp, posp, wpT, wcT, degsT)


def _final_body(acc_ref, g_ref, degs_ref, bc_ref, out_ref):
    deg = degs_ref[:, 0] + degs_ref[:, 1] + 1.0
    dinv = lax.rsqrt(deg)[:, None]
    bc = bc_ref[...]
    out_ref[:, :HALF] = (acc_ref[0] + g_ref[0]) * dinv + bc[:, :HALF]
    out_ref[:, HALF:] = (acc_ref[1] + g_ref[1]) * dinv + bc[:, HALF:]


def _tc_final(acc, g, degsT, bc2):
    return pl.pallas_call(
        _final_body,
        grid=(GRID,),
        in_specs=[
            pl.BlockSpec((NC, BN, HALF), lambda i: (0, i, 0)),
            pl.BlockSpec((NC, BN, HALF), lambda i: (0, i, 0)),
            pl.BlockSpec((BN, NC), lambda i: (i, 0)),
            pl.BlockSpec((1, CH), lambda i: (0, 0)),
        ],
        out_specs=pl.BlockSpec((BN, CH), lambda i: (i, 0)),
        out_shape=jax.ShapeDtypeStruct((NPAD, CH), jnp.float32),
    )(acc, g, degsT, bc2)


def kernel(x, edge_index, pos_embedding, Wp, Wc, bc):
    src = edge_index[0]
    dst = edge_index[1]
    npad = EPAD - E
    pad_ids = jnp.arange(npad, dtype=jnp.int32)
    srcp = jnp.concatenate([src, pad_ids % N]).reshape(NS, NCH, CHUNK)
    dstp = jnp.concatenate([dst, N + pad_ids % (R - N)]).reshape(NS, NCH, CHUNK)
    srcp2 = jnp.stack([srcp, srcp + NPAD])

    zeros1 = jnp.zeros((RPT,), jnp.float32)
    ones = jnp.ones((CHUNK,), jnp.float32)
    zrows = jnp.zeros((RPT, HALF), jnp.float32)

    degp = _sc_degree(dstp, zeros1, ones)
    degsT = degp.T

    xp = jnp.zeros((NPAD, CH), x.dtype).at[:N].set(x)
    posp = jnp.zeros((NPAD, POS), pos_embedding.dtype).at[:N].set(pos_embedding)
    g = _tc_dense(xp, posp, Wp.T, Wc.T, degsT)
    g2 = g.reshape(NC * NPAD, HALF)

    acc = _sc_messages(g2, srcp2, dstp, zrows)

    out = _tc_final(acc, g, degsT, bc.reshape(1, CH))
    return out[:N]


# speedup vs baseline: 26.4940x; 1.0377x over previous
import jax
import jax.numpy as jnp
from jax import lax
from jax.experimental import pallas as pl
from jax.experimental.pallas import tpu as pltpu
from jax.experimental.pallas import tpu_sc as plsc

N = 10000
E = 320000
CH = 256
HALF = 128
POS = 8

NC = 2
NS = 16
CHUNK = 128
NCH = 160
EPAD = NS * NCH * CHUNK
R = 10240
RPT = R // NS
DCH = NCH // NC

BN = 1024
NPAD = R
GRID = NPAD // BN


def _deg_body(dstp_hbm, zeros1_hbm, ones_hbm, degp_hbm, dst_v, ones_v, deg_sp, sem):
    c = lax.axis_index("c")
    s = lax.axis_index("s")
    pltpu.sync_copy(zeros1_hbm, deg_sp.at[pl.ds(s * RPT, RPT)])
    pltpu.sync_copy(ones_hbm, ones_v)
    pltpu.sync_copy(dstp_hbm.at[s, pl.ds(c * DCH, DCH)], dst_v)
    plsc.subcore_barrier()

    def body(j, carry):
        pltpu.sync_copy(ones_v, deg_sp.at[dst_v.at[j]], add=True)
        return carry

    lax.fori_loop(0, DCH, body, 0)
    plsc.subcore_barrier()
    pltpu.sync_copy(deg_sp.at[pl.ds(s * RPT, RPT)],
                    degp_hbm.at[c, pl.ds(s * RPT, RPT)])


def _sc_degree(dstp, zeros1, ones):
    mesh = plsc.VectorSubcoreMesh(core_axis_name="c", subcore_axis_name="s")
    return pl.kernel(
        _deg_body,
        out_type=jax.ShapeDtypeStruct((NC, R), jnp.float32),
        mesh=mesh,
        scratch_types=[
            pltpu.VMEM((DCH, CHUNK), jnp.int32),
            pltpu.VMEM((CHUNK,), jnp.float32),
            pltpu.VMEM_SHARED((R,), jnp.float32),
            pltpu.SemaphoreType.DMA,
        ],
    )(dstp, zeros1, ones)


def _msg_body(g2_hbm, srcp_hbm, dstp_hbm, zrows_hbm, acc_hbm,
              src_0, src_1, src_2, src_3, dst_0, dst_1, dst_2, dst_3,
              rows_a, rows_b, acc_sp,
              sem_i0, sem_i1, sem_i2, sem_i3,
              sem_ra, sem_rb, sem_sa, sem_sb):
    c = lax.axis_index("c")
    s = lax.axis_index("s")
    pltpu.sync_copy(zrows_hbm, acc_sp.at[pl.ds(s * RPT, RPT)])
    plsc.subcore_barrier()

    srcs = (src_0, src_1, src_2, src_3)
    dsts = (dst_0, dst_1, dst_2, dst_3)
    rows = (rows_a, rows_b)
    isems = (sem_i0, sem_i1, sem_i2, sem_i3)
    rsems = (sem_ra, sem_rb)
    ssems = (sem_sa, sem_sb)

    def fetch_idx(chunk, slot):
        pltpu.async_copy(srcp_hbm.at[c, s, chunk], srcs[slot], isems[slot])
        pltpu.async_copy(dstp_hbm.at[s, chunk], dsts[slot], isems[slot])

    def wait_idx(slot):
        pltpu.make_async_copy(srcp_hbm.at[c, s, 0], srcs[slot], isems[slot]).wait()
        pltpu.make_async_copy(dstp_hbm.at[s, 0], dsts[slot], isems[slot]).wait()

    def wait_rows(slot):
        pltpu.make_async_copy(g2_hbm.at[srcs[0]], rows[slot], rsems[slot]).wait()

    def wait_scat(slot):
        pltpu.make_async_copy(rows[slot], acc_sp.at[dsts[0]], ssems[slot]).wait()

    fetch_idx(0, 0)
    fetch_idx(1, 1)
    wait_idx(0)
    pltpu.async_copy(g2_hbm.at[src_0], rows_a, sem_ra)
    fetch_idx(2, 2)

    def outer(j0, carry):
        for q in range(4):
            j = j0 * 4 + q
            b = q % 2
            nb = 1 - b
            wait_idx((q + 1) % 4)
            wait_rows(b)

            @pl.when(j > 0)
            def _():
                wait_scat(nb)

            jn = jnp.minimum(j + 1, NCH - 1)
            del jn
            pltpu.async_copy(g2_hbm.at[srcs[(q + 1) % 4]], rows[nb], rsems[nb])
            pltpu.async_copy(rows[b], acc_sp.at[dsts[q]], ssems[b], add=True)
            fetch_idx(jnp.minimum(j + 3, NCH - 1), (q + 3) % 4)
        return carry

    lax.fori_loop(0, NCH // 4, outer, 0)
    wait_rows(0)
    wait_scat(1)
    wait_idx(1)
    wait_idx(2)

    plsc.subcore_barrier()
    pltpu.sync_copy(acc_sp.at[pl.ds(s * RPT, RPT)],
                    acc_hbm.at[c, pl.ds(s * RPT, RPT)])


def _sc_messages(g2, srcp2, dstp, zrows):
    mesh = plsc.VectorSubcoreMesh(core_axis_name="c", subcore_axis_name="s")
    return pl.kernel(
        _msg_body,
        out_type=jax.ShapeDtypeStruct((NC, R, HALF), jnp.float32),
        mesh=mesh,
        scratch_types=(
            [pltpu.VMEM((CHUNK,), jnp.int32)] * 8
            + [
                pltpu.VMEM((CHUNK, HALF), jnp.float32),
                pltpu.VMEM((CHUNK, HALF), jnp.float32),
                pltpu.VMEM_SHARED((R, HALF), jnp.float32),
            ]
            + [pltpu.SemaphoreType.DMA] * 8
        ),
    )(g2, srcp2, dstp, zrows)


def _dense_body(x_ref, pos_ref, wpT_ref, wcT_ref, degs_ref, g_ref):
    y = jnp.dot(x_ref[...], wpT_ref[...], preferred_element_type=jnp.float32)
    pos = pos_ref[...]
    acc = jnp.zeros((BN, CH), dtype=jnp.float32)
    for p in range(POS):
        acc = acc + y[:, p * CH:(p + 1) * CH] * pos[:, p:p + 1]
    h3 = jnp.dot(acc.astype(jnp.bfloat16), wcT_ref[...],
                 preferred_element_type=jnp.float32)
    deg = degs_ref[:, 0] + degs_ref[:, 1] + 1.0
    dinv = lax.rsqrt(deg)
    g = h3 * dinv[:, None]
    g_ref[0] = g[:, :HALF]
    g_ref[1] = g[:, HALF:]


def _tc_dense(xp, posp, wpT, wcT, degsT):
    return pl.pallas_call(
        _dense_body,
        grid=(GRID,),
        in_specs=[
            pl.BlockSpec((BN, CH), lambda i: (i, 0)),
            pl.BlockSpec((BN, POS), lambda i: (i, 0)),
            pl.BlockSpec((CH, CH * POS), lambda i: (0, 0)),
            pl.BlockSpec((CH, CH), lambda i: (0, 0)),
            pl.BlockSpec((BN, NC), lambda i: (i, 0)),
        ],
        out_specs=pl.BlockSpec((NC, BN, HALF), lambda i: (0, i, 0)),
        out_shape=jax.ShapeDtypeStruct((NC, N, HALF), jnp.float32),
    )(xp, posp, wpT, wcT, degsT)


def _final_body(acc_ref, g_ref, degs_ref, bc_ref, out_ref):
    deg = degs_ref[:, 0] + degs_ref[:, 1] + 1.0
    dinv = lax.rsqrt(deg)[:, None]
    bc = bc_ref[...]
    out_ref[:, :HALF] = (acc_ref[0] + g_ref[0]) * dinv + bc[:, :HALF]
    out_ref[:, HALF:] = (acc_ref[1] + g_ref[1]) * dinv + bc[:, HALF:]


def _tc_final(acc, g, degsT, bc2):
    return pl.pallas_call(
        _final_body,
        grid=(GRID,),
        in_specs=[
            pl.BlockSpec((NC, BN, HALF), lambda i: (0, i, 0)),
            pl.BlockSpec((NC, BN, HALF), lambda i: (0, i, 0)),
            pl.BlockSpec((BN, NC), lambda i: (i, 0)),
            pl.BlockSpec((1, CH), lambda i: (0, 0)),
        ],
        out_specs=pl.BlockSpec((BN, CH), lambda i: (i, 0)),
        out_shape=jax.ShapeDtypeStruct((N, CH), jnp.float32),
    )(acc, g, degsT, bc2)


def kernel(x, edge_index, pos_embedding, Wp, Wc, bc):
    src = edge_index[0]
    dst = edge_index[1]
    npad = EPAD - E
    pad_ids = jnp.arange(npad, dtype=jnp.int32)
    srcp = jnp.concatenate([src, pad_ids % N]).reshape(NS, NCH, CHUNK)
    dstp = jnp.concatenate([dst, N + pad_ids % (R - N)]).reshape(NS, NCH, CHUNK)
    srcp2 = jnp.stack([srcp, srcp + N])

    zeros1 = jnp.zeros((RPT,), jnp.float32)
    ones = jnp.ones((CHUNK,), jnp.float32)
    zrows = jnp.zeros((RPT, HALF), jnp.float32)

    degp = _sc_degree(dstp, zeros1, ones)
    degsT = degp[:, :N].T

    xp = x.astype(jnp.bfloat16)
    g = _tc_dense(xp, pos_embedding, Wp.T, Wc.T, degsT)
    g2 = g.reshape(NC * N, HALF)

    acc = _sc_messages(g2, srcp2, dstp, zrows)

    out = _tc_final(acc, g, degsT, bc.reshape(1, CH))
    return out
